# baseline (speedup 1.0000x reference)
import functools

import jax
import jax.numpy as jnp
from jax import lax
from jax.experimental import pallas as pl
from jax.experimental.pallas import tpu as pltpu
from jax.experimental.pallas import tpu_sc as plsc

VOCAB_SIZE = 8192
N_ROWS = 8192
NUM_CORES = 2
NUM_SUBCORES = 16
NUM_WORKERS = NUM_CORES * NUM_SUBCORES
ROWS_PER_WORKER = N_ROWS // NUM_WORKERS
LANES = 16
GROUPS = ROWS_PER_WORKER // LANES
ZERO_BLOCK_ROWS = 256


def _zero_body(out_ref):
    out_ref[...] = jnp.zeros((ZERO_BLOCK_ROWS, VOCAB_SIZE), jnp.float32)


def _tc_zeros():
    return pl.pallas_call(
        _zero_body,
        grid=(N_ROWS // ZERO_BLOCK_ROWS,),
        out_specs=pl.BlockSpec((ZERO_BLOCK_ROWS, VOCAB_SIZE), lambda i: (i, 0)),
        out_shape=jax.ShapeDtypeStruct((N_ROWS, VOCAB_SIZE), jnp.float32),
    )()


def _scatter_body(ids_hbm, out_hbm, ids_v, idx0, idx1, vals, sem):
    wid = lax.axis_index("s") * NUM_CORES + lax.axis_index("c")
    base_row = wid * ROWS_PER_WORKER

    pltpu.sync_copy(ids_hbm.at[pl.ds(base_row, ROWS_PER_WORKER)], ids_v)

    lane = lax.broadcasted_iota(jnp.int32, (LANES,), 0)
    fives = jnp.full((LANES,), 5.0, jnp.float32)
    for g in range(GROUPS):
        vec = ids_v[pl.ds(g * LANES, LANES)]
        col = lax.rem(vec, VOCAB_SIZE)
        flat = (base_row + g * LANES + lane) * VOCAB_SIZE + col
        if g < GROUPS // 2:
            idx0[pl.ds(g * LANES, LANES)] = flat
            vals[pl.ds(g * LANES, LANES)] = fives
        else:
            idx1[pl.ds((g - GROUPS // 2) * LANES, LANES)] = flat

    pltpu.async_copy(vals, out_hbm.at[idx0], sem).wait()
    pltpu.async_copy(vals, out_hbm.at[idx1], sem).wait()


_sc_scatter = functools.partial(
    pl.kernel,
    out_type=(),
    mesh=plsc.VectorSubcoreMesh(core_axis_name="c", subcore_axis_name="s"),
    scratch_types=[
        pltpu.VMEM((ROWS_PER_WORKER,), jnp.int32),
        pltpu.VMEM((ROWS_PER_WORKER // 2,), jnp.int32),
        pltpu.VMEM((ROWS_PER_WORKER // 2,), jnp.int32),
        pltpu.VMEM((ROWS_PER_WORKER // 2,), jnp.float32),
        pltpu.SemaphoreType.DMA,
    ],
)(_scatter_body)


def kernel(input_ids):
    bs, seq = input_ids.shape
    out_ref = jax.new_ref(_tc_zeros().reshape(-1))
    _sc_scatter(input_ids.reshape(-1), out_ref)
    return out_ref[...].reshape(bs, seq, VOCAB_SIZE)


# speedup vs baseline: 1.4916x; 1.4916x over previous
import functools

import jax
import jax.numpy as jnp
from jax import lax
from jax.experimental import pallas as pl
from jax.experimental.pallas import tpu as pltpu
from jax.experimental.pallas import tpu_sc as plsc

VOCAB_SIZE = 8192
N_ROWS = 8192
TOTAL_ELEMS = N_ROWS * VOCAB_SIZE
NUM_CORES = 2
NUM_SUBCORES = 16
NUM_WORKERS = NUM_CORES * NUM_SUBCORES
ROWS_PER_WORKER = N_ROWS // NUM_WORKERS
LANES = 16
GROUPS = ROWS_PER_WORKER // LANES

ZBUF_ELEMS = 1048576
NUM_ZERO_CHUNKS = TOTAL_ELEMS // ZBUF_ELEMS


def _tc_zero_body(out_hbm, zbuf, sem):
    zbuf[...] = jnp.zeros((ZBUF_ELEMS,), jnp.float32)
    copies = []
    for c in range(NUM_ZERO_CHUNKS):
        copies.append(
            pltpu.async_copy(
                zbuf, out_hbm.at[pl.ds(c * ZBUF_ELEMS, ZBUF_ELEMS)], sem
            )
        )
    for cp in copies:
        cp.wait()


def _scatter_body(ids_hbm, out_hbm, ids_v, idx0, idx1, vals, sem):
    wid = lax.axis_index("s") * NUM_CORES + lax.axis_index("c")
    base_row = wid * ROWS_PER_WORKER

    pltpu.sync_copy(ids_hbm.at[pl.ds(base_row, ROWS_PER_WORKER)], ids_v)

    lane = lax.broadcasted_iota(jnp.int32, (LANES,), 0)
    fives = jnp.full((LANES,), 5.0, jnp.float32)
    for g in range(GROUPS):
        vec = ids_v[pl.ds(g * LANES, LANES)]
        col = lax.rem(vec, VOCAB_SIZE)
        flat = (base_row + g * LANES + lane) * VOCAB_SIZE + col
        if g < GROUPS // 2:
            idx0[pl.ds(g * LANES, LANES)] = flat
            vals[pl.ds(g * LANES, LANES)] = fives
        else:
            idx1[pl.ds((g - GROUPS // 2) * LANES, LANES)] = flat

    pltpu.async_copy(vals, out_hbm.at[idx0], sem).wait()
    pltpu.async_copy(vals, out_hbm.at[idx1], sem).wait()


@functools.cache
def _build_kernels():
    tc_zeros = pl.kernel(
        _tc_zero_body,
        out_type=(),
        mesh=pltpu.create_tensorcore_mesh("core"),
        scratch_types=[
            pltpu.VMEM((ZBUF_ELEMS,), jnp.float32),
            pltpu.SemaphoreType.DMA,
        ],
    )
    sc_scatter = pl.kernel(
        _scatter_body,
        out_type=(),
        mesh=plsc.VectorSubcoreMesh(core_axis_name="c", subcore_axis_name="s"),
        scratch_types=[
            pltpu.VMEM((ROWS_PER_WORKER,), jnp.int32),
            pltpu.VMEM((ROWS_PER_WORKER // 2,), jnp.int32),
            pltpu.VMEM((ROWS_PER_WORKER // 2,), jnp.int32),
            pltpu.VMEM((ROWS_PER_WORKER // 2,), jnp.float32),
            pltpu.SemaphoreType.DMA,
        ],
    )
    return tc_zeros, sc_scatter


def kernel(input_ids):
    bs, seq = input_ids.shape
    tc_zeros, sc_scatter = _build_kernels()
    out_ref = jax.new_ref(lax.empty((TOTAL_ELEMS,), jnp.float32))
    tc_zeros(out_ref)
    sc_scatter(input_ids.reshape(-1), out_ref)
    return out_ref[...].reshape(bs, seq, VOCAB_SIZE)


# speedup vs baseline: 1.4956x; 1.0027x over previous
import functools

import jax
import jax.numpy as jnp
from jax import lax
from jax.experimental import pallas as pl
from jax.experimental.pallas import tpu as pltpu
from jax.experimental.pallas import tpu_sc as plsc

VOCAB_SIZE = 8192
N_ROWS = 8192
TOTAL_ELEMS = N_ROWS * VOCAB_SIZE
NUM_CORES = 2
NUM_SUBCORES = 16
NUM_WORKERS = NUM_CORES * NUM_SUBCORES
ROWS_PER_WORKER = N_ROWS // NUM_WORKERS
LANES = 16
GROUPS = ROWS_PER_WORKER // LANES

ZERO_BLOCK = 2097152
NUM_ZERO_BLOCKS = TOTAL_ELEMS // ZERO_BLOCK


def _tc_zero_body(out_hbm):
    def inner(out_blk):
        out_blk[...] = jnp.zeros((ZERO_BLOCK,), jnp.float32)

    pltpu.emit_pipeline(
        inner,
        grid=(NUM_ZERO_BLOCKS,),
        out_specs=[pl.BlockSpec((ZERO_BLOCK,), lambda i: (i,))],
    )(out_hbm)


def _scatter_body(ids_hbm, out_hbm, ids_v, idx0, idx1, vals, sem):
    wid = lax.axis_index("s") * NUM_CORES + lax.axis_index("c")
    base_row = wid * ROWS_PER_WORKER

    pltpu.sync_copy(ids_hbm.at[pl.ds(base_row, ROWS_PER_WORKER)], ids_v)

    lane = lax.broadcasted_iota(jnp.int32, (LANES,), 0)
    fives = jnp.full((LANES,), 5.0, jnp.float32)
    for g in range(GROUPS):
        vec = ids_v[pl.ds(g * LANES, LANES)]
        col = vec & (VOCAB_SIZE - 1)
        flat = (base_row + g * LANES + lane) * VOCAB_SIZE + col
        if g < GROUPS // 2:
            idx0[pl.ds(g * LANES, LANES)] = flat
            vals[pl.ds(g * LANES, LANES)] = fives
        else:
            idx1[pl.ds((g - GROUPS // 2) * LANES, LANES)] = flat

    pltpu.async_copy(vals, out_hbm.at[idx0], sem).wait()
    pltpu.async_copy(vals, out_hbm.at[idx1], sem).wait()


@functools.cache
def _build_kernels():
    tc_zeros = pl.kernel(
        _tc_zero_body,
        out_type=(),
        mesh=pltpu.create_tensorcore_mesh("core"),
    )
    sc_scatter = pl.kernel(
        _scatter_body,
        out_type=(),
        mesh=plsc.VectorSubcoreMesh(core_axis_name="c", subcore_axis_name="s"),
        scratch_types=[
            pltpu.VMEM((ROWS_PER_WORKER,), jnp.int32),
            pltpu.VMEM((ROWS_PER_WORKER // 2,), jnp.int32),
            pltpu.VMEM((ROWS_PER_WORKER // 2,), jnp.int32),
            pltpu.VMEM((ROWS_PER_WORKER // 2,), jnp.float32),
            pltpu.SemaphoreType.DMA,
        ],
    )
    return tc_zeros, sc_scatter


def kernel(input_ids):
    bs, seq = input_ids.shape
    tc_zeros, sc_scatter = _build_kernels()
    out_ref = jax.new_ref(lax.empty((TOTAL_ELEMS,), jnp.float32))
    tc_zeros(out_ref)
    sc_scatter(input_ids.reshape(-1), out_ref)
    return out_ref[...].reshape(bs, seq, VOCAB_SIZE)


# speedup vs baseline: 1.5059x; 1.0069x over previous
import functools

import jax
import jax.numpy as jnp
from jax import lax
from jax.experimental import pallas as pl
from jax.experimental.pallas import tpu as pltpu
from jax.experimental.pallas import tpu_sc as plsc

VOCAB_SIZE = 8192
N_ROWS = 8192
TOTAL_ELEMS = N_ROWS * VOCAB_SIZE
NUM_CORES = 2
NUM_SUBCORES = 16
NUM_WORKERS = NUM_CORES * NUM_SUBCORES
ROWS_PER_WORKER = N_ROWS // NUM_WORKERS
LANES = 16
GROUPS = ROWS_PER_WORKER // LANES

SUB = 128
SUBS_PER_ROW = VOCAB_SIZE // SUB
TOTAL_SUBS = TOTAL_ELEMS // SUB
ZERO_BLOCK_SUBS = 16384
NUM_ZERO_BLOCKS = TOTAL_SUBS // ZERO_BLOCK_SUBS


def _tc_zero_body(out_hbm):
    def inner(out_blk):
        out_blk[...] = jnp.zeros((ZERO_BLOCK_SUBS, SUB), jnp.float32)

    pltpu.emit_pipeline(
        inner,
        grid=(NUM_ZERO_BLOCKS,),
        out_specs=[pl.BlockSpec((ZERO_BLOCK_SUBS, SUB), lambda i: (i, 0))],
    )(out_hbm)


def _scatter_body(ids_hbm, out_2d, ids_v, buf0, buf1, idx0, idx1, sem):
    wid = lax.axis_index("s") * NUM_CORES + lax.axis_index("c")
    base_row = wid * ROWS_PER_WORKER

    pltpu.sync_copy(ids_hbm.at[pl.ds(base_row, ROWS_PER_WORKER)], ids_v)

    lane = lax.broadcasted_iota(jnp.int32, (LANES,), 0)
    half = GROUPS // 2

    for g in range(GROUPS):
        vec = ids_v[pl.ds(g * LANES, LANES)]
        col = vec & (VOCAB_SIZE - 1)
        sub_idx = (base_row + g * LANES + lane) * SUBS_PER_ROW + (col >> 7)
        if g < half:
            idx0[pl.ds(g * LANES, LANES)] = sub_idx
        else:
            idx1[pl.ds((g - half) * LANES, LANES)] = sub_idx

    def row_body(k, carry):
        g16 = (k >> 4) << 4
        vec = ids_v[pl.ds(g16, LANES)]
        inner = vec & (SUB - 1)
        bcast = inner.at[jnp.full((LANES,), k & (LANES - 1), jnp.int32)].get(
            mode="promise_in_bounds"
        )
        kb = k & (ROWS_PER_WORKER // 2 - 1)
        for u in range(SUB // LANES):
            val = jnp.where(lane + u * LANES == bcast, 5.0, 0.0).astype(jnp.float32)
            pl.when(k < ROWS_PER_WORKER // 2)(
                lambda v=val, uu=u: buf0.at[kb, pl.ds(uu * LANES, LANES)].set(v)
            )
            pl.when(k >= ROWS_PER_WORKER // 2)(
                lambda v=val, uu=u: buf1.at[kb, pl.ds(uu * LANES, LANES)].set(v)
            )
        return carry

    lax.fori_loop(0, ROWS_PER_WORKER, row_body, 0)

    pltpu.async_copy(buf0, out_2d.at[idx0], sem).wait()
    pltpu.async_copy(buf1, out_2d.at[idx1], sem).wait()


@functools.cache
def _build_kernels():
    tc_zeros = pl.kernel(
        _tc_zero_body,
        out_type=(),
        mesh=pltpu.create_tensorcore_mesh("core"),
    )
    sc_scatter = pl.kernel(
        _scatter_body,
        out_type=(),
        mesh=plsc.VectorSubcoreMesh(core_axis_name="c", subcore_axis_name="s"),
        scratch_types=[
            pltpu.VMEM((ROWS_PER_WORKER,), jnp.int32),
            pltpu.VMEM((ROWS_PER_WORKER // 2, SUB), jnp.float32),
            pltpu.VMEM((ROWS_PER_WORKER // 2, SUB), jnp.float32),
            pltpu.VMEM((ROWS_PER_WORKER // 2,), jnp.int32),
            pltpu.VMEM((ROWS_PER_WORKER // 2,), jnp.int32),
            pltpu.SemaphoreType.DMA,
        ],
    )
    return tc_zeros, sc_scatter


def kernel(input_ids):
    bs, seq = input_ids.shape
    tc_zeros, sc_scatter = _build_kernels()
    out_ref = jax.new_ref(lax.empty((TOTAL_SUBS, SUB), jnp.float32))
    tc_zeros(out_ref)
    sc_scatter(input_ids.reshape(-1), out_ref)
    return out_ref[...].reshape(bs, seq, VOCAB_SIZE)


# speedup vs baseline: 5.3537x; 3.5552x over previous
import functools

import jax
import jax.numpy as jnp
from jax import lax
from jax.experimental import pallas as pl
from jax.experimental.pallas import tpu as pltpu
from jax.experimental.pallas import tpu_sc as plsc

VOCAB_SIZE = 8192
N_ROWS = 8192
NUM_CORES = 2
NUM_SUBCORES = 16
NUM_WORKERS = NUM_CORES * NUM_SUBCORES
ROWS_PER_WORKER = N_ROWS // NUM_WORKERS
LANES = 16
GROUPS = ROWS_PER_WORKER // LANES

SUB = 128
BLOCK_ROWS = 256


def _sc_payload_body(ids_hbm, subrows_hbm, subpos_hbm, ids_v, buf, spbuf, lock):
    del lock
    wid = lax.axis_index("s") * NUM_CORES + lax.axis_index("c")
    base_row = wid * ROWS_PER_WORKER

    pltpu.sync_copy(ids_hbm.at[pl.ds(base_row, ROWS_PER_WORKER)], ids_v)

    lane = lax.broadcasted_iota(jnp.int32, (LANES,), 0)

    for g in range(GROUPS):
        vec = ids_v[pl.ds(g * LANES, LANES)]
        col = vec & (VOCAB_SIZE - 1)
        spbuf[pl.ds(g * LANES, LANES)] = col >> 7

    def row_body(k, carry):
        g16 = (k >> 4) << 4
        vec = ids_v[pl.ds(g16, LANES)]
        inner = vec & (SUB - 1)
        bcast = inner.at[jnp.full((LANES,), k & (LANES - 1), jnp.int32)].get(
            mode="promise_in_bounds"
        )
        for u in range(SUB // LANES):
            val = jnp.where(lane + u * LANES == bcast, 5.0, 0.0).astype(jnp.float32)
            buf[k, pl.ds(u * LANES, LANES)] = val
        return carry

    lax.fori_loop(0, ROWS_PER_WORKER, row_body, 0)

    pltpu.sync_copy(buf, subrows_hbm.at[pl.ds(base_row, ROWS_PER_WORKER)])
    pltpu.sync_copy(spbuf, subpos_hbm.at[pl.ds(base_row, ROWS_PER_WORKER)])


@functools.cache
def _build_sc_payload():
    return pl.kernel(
        _sc_payload_body,
        out_type=(
            jax.ShapeDtypeStruct((N_ROWS, SUB), jnp.float32),
            jax.ShapeDtypeStruct((N_ROWS,), jnp.int32),
        ),
        mesh=plsc.VectorSubcoreMesh(core_axis_name="c", subcore_axis_name="s"),
        scratch_types=[
            pltpu.VMEM((ROWS_PER_WORKER,), jnp.int32),
            pltpu.VMEM((ROWS_PER_WORKER, SUB), jnp.float32),
            pltpu.VMEM((ROWS_PER_WORKER,), jnp.int32),
            pltpu.SemaphoreType.DMA,
        ],
    )


def _tc_assemble_body(subpos_ref, subrows_ref, out_ref):
    out_ref[...] = jnp.zeros((BLOCK_ROWS, VOCAB_SIZE), jnp.float32)
    for r in range(BLOCK_ROWS):
        c = subpos_ref[0, 0, r]
        start = pl.multiple_of(c * SUB, SUB)
        out_ref[r, pl.ds(start, SUB)] = subrows_ref[r, :]


def kernel(input_ids):
    bs, seq = input_ids.shape
    subrows, subpos = _build_sc_payload()(input_ids.reshape(-1))
    out = pl.pallas_call(
        _tc_assemble_body,
        grid=(N_ROWS // BLOCK_ROWS,),
        in_specs=[
            pl.BlockSpec(
                (1, 1, BLOCK_ROWS),
                lambda i: (i, 0, 0),
                memory_space=pltpu.SMEM,
            ),
            pl.BlockSpec((BLOCK_ROWS, SUB), lambda i: (i, 0)),
        ],
        out_specs=pl.BlockSpec((BLOCK_ROWS, VOCAB_SIZE), lambda i: (i, 0)),
        out_shape=jax.ShapeDtypeStruct((N_ROWS, VOCAB_SIZE), jnp.float32),
    )(subpos.reshape(N_ROWS // BLOCK_ROWS, 1, BLOCK_ROWS), subrows)
    return out.reshape(bs, seq, VOCAB_SIZE)
